# baseline (speedup 1.0000x reference)
import functools

import jax
import jax.numpy as jnp
from jax import lax
from jax.experimental import pallas as pl
from jax.experimental.pallas import tpu as pltpu
from jax.experimental.pallas import tpu_sc as plsc

N = 50000
E = 800000
E_PAD = 819200
IN_CH = 32
EMB = 64
GW = 128
NB = 1000
BE = 2048

_mesh = lambda: plsc.VectorSubcoreMesh(core_axis_name="c", subcore_axis_name="s")


def _sc_gather(table, idx, width):
  @functools.partial(
      pl.kernel,
      out_type=jax.ShapeDtypeStruct((E_PAD, width), jnp.float32),
      mesh=_mesh(),
  )
  def k(t_hbm, i_hbm, o_hbm):
    def body(i_vmem, o_vmem):
      pltpu.sync_copy(t_hbm.at[i_vmem.at[0]], o_vmem)

    pltpu.emit_pipeline(
        body,
        grid=(E_PAD // GW,),
        in_specs=[pl.BlockSpec((1, GW), lambda i: (0, i))],
        out_specs=[pl.BlockSpec((GW, width), lambda i: (i, 0))],
        core_axis_name=("c", "s"),
        dimension_semantics=(pltpu.PARALLEL,),
    )(i_hbm, o_hbm)

  return k(table, idx)


ESB = 2000


def _segsum(upd, dst3d, width):
  def body(dst_ref, u_ref, o_ref):
    @pl.when(pl.program_id(0) == 0)
    def _():
      o_ref[...] = jnp.zeros_like(o_ref)

    def step(j, c):
      d = dst_ref[0, 0, j]
      o_ref[pl.ds(d, 1), :] = o_ref[pl.ds(d, 1), :] + u_ref[pl.ds(j, 1), :]
      return c

    jax.lax.fori_loop(0, ESB, step, 0, unroll=True)

  return pl.pallas_call(
      body,
      grid=(E // ESB,),
      in_specs=[
          pl.BlockSpec((1, 1, ESB), lambda i: (i, 0, 0),
                       memory_space=pltpu.SMEM),
          pl.BlockSpec((ESB, width), lambda i: (i, 0)),
      ],
      out_specs=pl.BlockSpec((N, width), lambda i: (0, 0)),
      out_shape=jax.ShapeDtypeStruct((N, width), jnp.float32),
  )(dst3d, upd)


def _leaky(m):
  return jnp.where(m >= 0, m, 0.2 * m)


def _self_logit(xl, xr, el, att):
  m = _leaky(xl + xr + el) * att
  c0 = jnp.sum(m[:, :32], axis=1, keepdims=True)
  c1 = jnp.sum(m[:, 32:64], axis=1, keepdims=True)
  return c0, c1


def _prep1_body(x_ref, la_ref, mw_ref, mb_ref, wl_ref, bl_ref, wr_ref,
                br_ref, we_ref, att_ref, xl_ref, xrc_ref):
  y = jnp.maximum(jnp.dot(x_ref[...], mw_ref[...],
                          preferred_element_type=jnp.float32) + mb_ref[...], 0.0)
  xl = jnp.dot(y, wl_ref[...], preferred_element_type=jnp.float32) + bl_ref[...]
  xr = jnp.dot(y, wr_ref[...], preferred_element_type=jnp.float32) + br_ref[...]
  la = la_ref[...]
  lattr = la[:, :13] / jnp.maximum(la[:, 13:14], 1.0)
  el = jnp.dot(lattr, we_ref[...], preferred_element_type=jnp.float32)
  c0, c1 = _self_logit(xl, xr, el, att_ref[...])
  zpad = jnp.zeros((xr.shape[0], 64), jnp.float32)
  xl_ref[...] = jnp.concatenate([xl, zpad], axis=1)
  xrc_ref[...] = jnp.concatenate([xr, c0, c1, zpad[:, :62]], axis=1)


def _prep2_body(y_ref, la_ref,
                wl1_ref, bl1_ref, wr1_ref, br1_ref, we1_ref, att1_ref,
                wl2_ref, bl2_ref, wr2_ref, br2_ref, we2_ref, att2_ref,
                xl1_ref, xrc1_ref, xl2_ref, xrc2_ref):
  y = y_ref[...]
  la = la_ref[...]
  lattr = la[:, :13] / jnp.maximum(la[:, 13:14], 1.0)
  zpad = jnp.zeros((y.shape[0], 64), jnp.float32)
  for wl_r, bl_r, wr_r, br_r, we_r, att_r, xl_o, xrc_o in (
      (wl1_ref, bl1_ref, wr1_ref, br1_ref, we1_ref, att1_ref, xl1_ref, xrc1_ref),
      (wl2_ref, bl2_ref, wr2_ref, br2_ref, we2_ref, att2_ref, xl2_ref, xrc2_ref)):
    xl = jnp.dot(y, wl_r[...], preferred_element_type=jnp.float32) + bl_r[...]
    xr = jnp.dot(y, wr_r[...], preferred_element_type=jnp.float32) + br_r[...]
    el = jnp.dot(lattr, we_r[...], preferred_element_type=jnp.float32)
    c0, c1 = _self_logit(xl, xr, el, att_r[...])
    xl_o[...] = jnp.concatenate([xl, zpad], axis=1)
    xrc_o[...] = jnp.concatenate([xr, c0, c1, zpad[:, :62]], axis=1)


def _edge_body(ea_ref, xs_ref, xd_ref, we_ref, att_ref, w_ref, ex_ref):
  xs = xs_ref[...][:, :64]
  xd = xd_ref[...]
  ef = jnp.dot(ea_ref[...], we_ref[...], preferred_element_type=jnp.float32)
  m = _leaky(xs + xd[:, :64] + ef) * att_ref[...]
  l0 = jnp.sum(m[:, :32], axis=1, keepdims=True)
  l1 = jnp.sum(m[:, 32:64], axis=1, keepdims=True)
  ex0 = jnp.exp(l0 - xd[:, 64:65])
  ex1 = jnp.exp(l1 - xd[:, 65:66])
  w_ref[...] = xs * jnp.concatenate(
      [jnp.broadcast_to(ex0, (xs.shape[0], 32)),
       jnp.broadcast_to(ex1, (xs.shape[0], 32))], axis=1)
  ex_ref[...] = jnp.concatenate(
      [ex0, ex1, jnp.zeros((xs.shape[0], 14), jnp.float32)], axis=1)


def _gat_out(accw, acce, xl, bias):
  num = accw + xl[:, :64]
  den0 = acce[:, 0:1] + 1.0
  den1 = acce[:, 1:2] + 1.0
  den = jnp.concatenate([jnp.broadcast_to(den0, (accw.shape[0], 32)),
                         jnp.broadcast_to(den1, (accw.shape[0], 32))], axis=1)
  return num / den + bias


def _finish_g1_body(accw_ref, acce_ref, xl_ref, bias_ref, ty_ref, hw_ref,
                    hb_ref, out_ref):
  g = _gat_out(accw_ref[...], acce_ref[...], xl_ref[...], bias_ref[...])
  ty = ty_ref[...]
  out = jnp.zeros((g.shape[0], 64), jnp.float32)
  for t in range(16):
    r = jnp.dot(g, hw_ref[t], preferred_element_type=jnp.float32) + hb_ref[t]
    out = out + jnp.where(ty == t, r, 0.0)
  out_ref[...] = out


def _finish_mulv_body(accw_ref, acce_ref, xl_ref, bias_ref, ty_ref, x_ref,
                      wc_ref, bc_ref, out_ref):
  g = _gat_out(accw_ref[...], acce_ref[...], xl_ref[...], bias_ref[...])
  x = x_ref[...]
  ty = ty_ref[...]
  out = jnp.zeros((g.shape[0], 64), jnp.float32)
  for t in range(16):
    wc = wc_ref[t]
    r = (jnp.dot(g, wc[:64, :], preferred_element_type=jnp.float32)
         + jnp.dot(x, wc[64:, :], preferred_element_type=jnp.float32)
         + bc_ref[t])
    out = out + jnp.where(ty == t, r, 0.0)
  out_ref[...] = out


def _combine_body(w1_ref, b1_ref, w2_ref, b2_ref, wc_ref, bc_ref):
  for t in range(16):
    w2 = w2_ref[t]
    wc_ref[t] = jnp.dot(w1_ref[t], w2, preferred_element_type=jnp.float32)
    bc_ref[t] = jnp.dot(b1_ref[t], w2, preferred_element_type=jnp.float32) + b2_ref[t]


def _full(shape):
  nd = len(shape)
  return pl.BlockSpec(shape, lambda *_: (0,) * nd)


def _tc_call(body, grid, in_specs, out_specs, out_shapes):
  return pl.pallas_call(
      body, grid=grid, in_specs=in_specs, out_specs=out_specs,
      out_shape=out_shapes)


def kernel(x, edge_index, edge_attr, atom_types, mlp_W, mlp_b,
           g1_Wl, g1_bl, g1_Wr, g1_br, g1_We, g1_att, g1_bias,
           gmu_Wl, gmu_bl, gmu_Wr, gmu_br, gmu_We, gmu_att, gmu_bias,
           glv_Wl, glv_bl, glv_Wr, glv_br, glv_We, glv_att, glv_bias,
           hs_W, hs_b, hmu1_W, hmu1_b, hmu2_W, hmu2_b,
           hlv1_W, hlv1_b, hlv2_W, hlv2_b):
  f32 = jnp.float32
  src = edge_index[0]
  dst = edge_index[1]
  pad_idx = (jnp.arange(E_PAD - E, dtype=jnp.int32) % N)
  src_p = jnp.concatenate([src, pad_idx]).reshape(1, E_PAD)
  dst_p = jnp.concatenate([dst, pad_idx]).reshape(1, E_PAD)
  ea_p = jnp.concatenate([edge_attr, jnp.zeros((E_PAD - E, 13), f32)], axis=0)
  dst3d = dst.reshape(E // ESB, 1, ESB)
  ea16 = jnp.concatenate(
      [edge_attr, jnp.ones((E, 1), f32), jnp.zeros((E, 2), f32)], axis=1)
  ty2 = atom_types.reshape(N, 1)
  r = lambda b: b.reshape(1, 64)
  att_row = lambda a: a.reshape(1, 64)

  loop_acc = _segsum(ea16, dst3d, 16)

  grid_n = (N // NB,)
  n_spec = lambda w: pl.BlockSpec((NB, w), lambda i: (i, 0))

  def gat_edge_phase(xl, xrc, we, att):
    xs = _sc_gather(xl, src_p, 128)
    xd = _sc_gather(xrc, dst_p, 128)
    upd_w, upd_e = _tc_call(
        _edge_body, (E_PAD // BE,),
        [pl.BlockSpec((BE, 13), lambda i: (i, 0)),
         pl.BlockSpec((BE, 128), lambda i: (i, 0)),
         pl.BlockSpec((BE, 128), lambda i: (i, 0)),
         _full((13, 64)), _full((1, 64))],
        [pl.BlockSpec((BE, 64), lambda i: (i, 0)),
         pl.BlockSpec((BE, 16), lambda i: (i, 0))],
        [jax.ShapeDtypeStruct((E_PAD, 64), f32),
         jax.ShapeDtypeStruct((E_PAD, 16), f32)],
    )(ea_p, xs, xd, we, att_row(att))
    acc_w = _segsum(upd_w, dst3d, 64)
    acc_e = _segsum(upd_e, dst3d, 16)
    return acc_w, acc_e

  xl1, xrc1 = _tc_call(
      _prep1_body, grid_n,
      [n_spec(IN_CH), n_spec(16), _full((IN_CH, 64)), _full((1, 64)),
       _full((64, 64)), _full((1, 64)), _full((64, 64)), _full((1, 64)),
       _full((13, 64)), _full((1, 64))],
      [n_spec(128), n_spec(128)],
      [jax.ShapeDtypeStruct((N, 128), f32),
       jax.ShapeDtypeStruct((N, 128), f32)],
  )(x, loop_acc, mlp_W, r(mlp_b), g1_Wl, r(g1_bl), g1_Wr, r(g1_br),
    g1_We, att_row(g1_att))
  accw1, acce1 = gat_edge_phase(xl1, xrc1, g1_We, g1_att)
  y2 = _tc_call(
      _finish_g1_body, grid_n,
      [n_spec(64), n_spec(16), n_spec(128), _full((1, 64)),
       pl.BlockSpec((NB, 1), lambda i: (i, 0)),
       _full((16, 64, 64)), _full((16, 1, 64))],
      n_spec(64),
      jax.ShapeDtypeStruct((N, 64), f32),
  )(accw1, acce1, xl1, r(g1_bias), ty2, hs_W, hs_b.reshape(16, 1, 64))

  xl_mu, xrc_mu, xl_lv, xrc_lv = _tc_call(
      _prep2_body, grid_n,
      [n_spec(64), n_spec(16),
       _full((64, 64)), _full((1, 64)), _full((64, 64)), _full((1, 64)),
       _full((13, 64)), _full((1, 64)),
       _full((64, 64)), _full((1, 64)), _full((64, 64)), _full((1, 64)),
       _full((13, 64)), _full((1, 64))],
      [n_spec(128), n_spec(128), n_spec(128), n_spec(128)],
      [jax.ShapeDtypeStruct((N, 128), f32),
       jax.ShapeDtypeStruct((N, 128), f32),
       jax.ShapeDtypeStruct((N, 128), f32),
       jax.ShapeDtypeStruct((N, 128), f32)],
  )(y2, loop_acc,
    gmu_Wl, r(gmu_bl), gmu_Wr, r(gmu_br), gmu_We, att_row(gmu_att),
    glv_Wl, r(glv_bl), glv_Wr, r(glv_br), glv_We, att_row(glv_att))

  accw_mu, acce_mu = gat_edge_phase(xl_mu, xrc_mu, gmu_We, gmu_att)
  accw_lv, acce_lv = gat_edge_phase(xl_lv, xrc_lv, glv_We, glv_att)

  def combine(w1, b1, w2, b2):
    return _tc_call(
        _combine_body, (),
        [_full((16, 96, 64)), _full((16, 1, 64)),
         _full((16, 64, 64)), _full((16, 1, 64))],
        [_full((16, 96, 64)), _full((16, 1, 64))],
        [jax.ShapeDtypeStruct((16, 96, 64), f32),
         jax.ShapeDtypeStruct((16, 1, 64), f32)],
    )(w1, b1.reshape(16, 1, 64), w2, b2.reshape(16, 1, 64))

  wc_mu, bc_mu = combine(hmu1_W, hmu1_b, hmu2_W, hmu2_b)
  wc_lv, bc_lv = combine(hlv1_W, hlv1_b, hlv2_W, hlv2_b)

  def finish(accw, acce, xl, bias, wc, bc):
    return _tc_call(
        _finish_mulv_body, grid_n,
        [n_spec(64), n_spec(16), n_spec(128), _full((1, 64)),
         pl.BlockSpec((NB, 1), lambda i: (i, 0)), n_spec(IN_CH),
         _full((16, 96, 64)), _full((16, 1, 64))],
        n_spec(64),
        jax.ShapeDtypeStruct((N, 64), f32),
    )(accw, acce, xl, r(bias), ty2, x, wc, bc)

  mu = finish(accw_mu, acce_mu, xl_mu, gmu_bias, wc_mu, bc_mu)
  lv = finish(accw_lv, acce_lv, xl_lv, glv_bias, wc_lv, bc_lv)
  return (mu, lv)
